# baseline (speedup 1.0000x reference)
import functools

import jax
import jax.numpy as jnp
from jax import lax
from jax.experimental import pallas as pl
from jax.experimental.pallas import tpu as pltpu
from jax.experimental.pallas import tpu_sc as plsc

D = 64
NC = 2
NS = 16
NW = NC * NS
CH = 128
L = 16
CW = 512
NE = 1000000
NFULL = NE // CW
TAIL0 = NFULL * CW
BPW = 1024
GPW = BPW // L
MAXG = NW * GPW
SEG = BPW + L
UNROLL = 8

_CP = pltpu.CompilerParams(needs_layout_passes=False, use_tc_tiling_on_sc=True)
_MESH = dict(core_axis_name="c", subcore_axis_name="s")


def _rsqrt(x):
    i = plsc.bitcast(x, jnp.int32)
    i = jnp.int32(0x5F3759DF) - (i >> 1)
    y = plsc.bitcast(i, jnp.float32)
    hx = x * jnp.float32(0.5)
    for _ in range(3):
        y = y * (jnp.float32(1.5) - hx * y * y)
    return y


def _sweep_body(sent_hbm, entT_hbm, tailT_hbm, mat_hbm,
                sent_v, bbuf, tbuf, stage, semb, semf):
    wid = lax.axis_index("s") * NC + lax.axis_index("c")
    g0 = wid * GPW

    pltpu.sync_copy(
        sent_hbm.at[pl.ds(pl.multiple_of(wid * BPW, 8), SEG)], sent_v)

    iota = lax.iota(jnp.int32, L)

    e_first = sent_v[pl.ds(0, L)][0]
    e_last = sent_v[pl.ds(BPW - L, L)][L - 1]
    c_lo = e_first >> 9
    c_hi_all = e_last >> 9
    c_hi = jnp.minimum(c_hi_all, NFULL - 1)
    nck = jnp.maximum(c_hi - c_lo + 1, 0)

    def flush_group(g, fg):
        @pl.when(fg >= 2)
        def _():
            pltpu.make_async_copy(stage.at[0], mat_hbm.at[0], semf).wait()
        pltpu.async_copy(stage.at[g & 3], mat_hbm.at[g0 + g], semf)

    def _drain(fg3):
        @pl.when(fg3 >= 1)
        def _():
            pltpu.make_async_copy(stage.at[0], mat_hbm.at[0], semf).wait()

        @pl.when(fg3 >= 2)
        def _():
            pltpu.make_async_copy(stage.at[0], mat_hbm.at[0], semf).wait()

    def extract_chunk(cid, buf, slot_s, start, wmax, p, fg):
        slotv = jnp.full((L,), slot_s, jnp.int32)

        def cond(state):
            return state[2]

        def body(state):
            p_, fg_, _ = state
            ev = sent_v[pl.ds(p_, L)]
            cnt = plsc.all_reduce_population_count((ev >> 9) == cid)[0]
            cv = jnp.clip(ev - start, 0, wmax)
            tv = p_ + iota
            ssv = (tv >> 4) & 3
            rowv = tv & (L - 1)

            @pl.when(cnt > 0)
            def _():
                def jstep(i, _):
                    for u in range(UNROLL):
                        jv = (jnp.full((L,), i * UNROLL + u, jnp.int32)
                              + iota) & (D - 1)
                        g = plsc.load_gather(buf, [slotv, jv, cv])
                        plsc.store_scatter(stage, [ssv, rowv, jv], g)
                    return 0

                lax.fori_loop(0, D // UNROLL, jstep, 0)

            newp = p_ + cnt

            def doflush(fg2):
                flush_group(fg2, fg2)
                return fg2 + 1

            fg_ = lax.cond((newp >> 4) > fg_, doflush, lambda f: f, fg_)
            return (newp, fg_, (cnt == L) & (newp < BPW))

        p, fg, _ = lax.while_loop(cond, body, (p, fg, True))
        return p, fg

    def start_chunk(k, slot):
        off = pl.multiple_of((c_lo + k) * CW, CW)
        return pltpu.async_copy(
            entT_hbm.at[:, pl.ds(off, CW)], bbuf.at[slot], semb)

    @pl.when(nck > 0)
    def _():
        start_chunk(0, 0)

    @pl.when(nck > 1)
    def _():
        start_chunk(1, 1)

    def chunk_body(k, carry):
        p, fg = carry
        pltpu.make_async_copy(
            entT_hbm.at[:, pl.ds(0, CW)], bbuf.at[0], semb).wait()

        @pl.when(k + 2 < nck)
        def _():
            start_chunk(k + 2, (k + 2) % 3)

        return extract_chunk(c_lo + k, bbuf, k % 3, (c_lo + k) * CW,
                             CW - 1, p, fg)

    p, fg = lax.fori_loop(0, nck, chunk_body,
                          (jnp.int32(0), jnp.int32(0)))

    @pl.when(c_hi_all >= NFULL)
    def _():
        pltpu.sync_copy(tailT_hbm, tbuf.at[0])
        p2, fg2 = extract_chunk(jnp.int32(NFULL), tbuf, 0, TAIL0,
                                CH - 1, p, fg)
        del p2
        _drain(fg2)

    @pl.when(c_hi_all < NFULL)
    def _():
        _drain(fg)


def _score_body(nch, hpos_hbm, tpos_hbm, rrow_hbm, rpar_hbm, mat_hbm,
                rel_hbm, out_hbm, hidx, tidx, ridx, rpar, hbuf, tbuf, rbuf,
                sbuf, sem):
    wid = lax.axis_index("s") * NC + lax.axis_index("c")
    base = wid * (nch * CH)

    pltpu.sync_copy(hpos_hbm.at[wid], hidx)
    pltpu.sync_copy(tpos_hbm.at[wid], tidx)
    pltpu.sync_copy(rrow_hbm.at[wid], ridx)
    pltpu.sync_copy(rpar_hbm.at[wid], rpar)

    iota = lax.iota(jnp.int32, L)

    def gather(c, slot):
        return (
            pltpu.async_copy(mat_hbm.at[hidx.at[c]], hbuf.at[slot], sem),
            pltpu.async_copy(mat_hbm.at[tidx.at[c]], tbuf.at[slot], sem),
            pltpu.async_copy(rel_hbm.at[ridx.at[c]], rbuf.at[slot], sem),
        )

    cps = gather(0, 0)
    for c in range(nch):
        slot = c % 2
        for cp in cps:
            cp.wait()
        if c + 1 < nch:
            cps = gather(c + 1, 1 - slot)

        for g in range(CH // L):
            row = iota + g * L
            pr = rpar[c, pl.ds(g * L, L)] << 6
            zeros = jnp.zeros((L,), jnp.float32)

            def step(i, acc):
                sh, sr, st, xhr, xht, xrt = acc
                for u in range(UNROLL):
                    jv = (jnp.full((L,), i * UNROLL + u, jnp.int32)
                          + iota) & (D - 1)
                    gh = plsc.load_gather(hbuf.at[slot], [row, jv])
                    gt = plsc.load_gather(tbuf.at[slot], [row, jv])
                    gr = plsc.load_gather(rbuf.at[slot], [row, pr + jv])
                    sh = sh + gh * gh
                    sr = sr + gr * gr
                    st = st + gt * gt
                    xhr = xhr + gh * gr
                    xht = xht + gh * gt
                    xrt = xrt + gr * gt
                return (sh, sr, st, xhr, xht, xrt)

            sh, sr, st, xhr, xht, xrt = lax.fori_loop(
                0, D // UNROLL, step, (zeros,) * 6)

            cap = jnp.full((L,), 1e12, jnp.float32)
            ih = jnp.minimum(_rsqrt(sh), cap)
            ir = jnp.minimum(_rsqrt(sr), cap)
            it = jnp.minimum(_rsqrt(st), cap)
            s2 = (sh * ih * ih + sr * ir * ir + st * it * it
                  + jnp.float32(2.0) * (xhr * ih * ir - xht * ih * it
                                        - xrt * ir * it))
            s2 = jnp.maximum(s2, jnp.float32(0.0))
            sbuf[pl.ds(c * CH + g * L, L)] = s2 * _rsqrt(s2)

    pltpu.sync_copy(sbuf, out_hbm.at[pl.ds(base, nch * CH)])


def kernel(head, relation, tail, entity_embeddings, relation_embeddings):
    b = head.shape[0]
    nch = b // (NW * CH)

    ent_all = jnp.concatenate([head, tail])
    order = jnp.argsort(ent_all)
    inv_order = jnp.argsort(order).astype(jnp.int32)
    sorted_ent = ent_all[order]
    hpos = inv_order[:b].reshape(NW, nch, CH)
    tpos = inv_order[b:].reshape(NW, nch, CH)

    sent_pad = jnp.concatenate(
        [sorted_ent.astype(jnp.int32), jnp.zeros((SEG + 8,), jnp.int32)])

    entT = entity_embeddings.T
    tailT = jnp.pad(entity_embeddings[TAIL0:], ((0, CH - (NE - TAIL0)),
                                                (0, 0))).T

    sweep = functools.partial(
        pl.kernel,
        out_type=jax.ShapeDtypeStruct((MAXG, L, CH), jnp.float32),
        mesh=plsc.VectorSubcoreMesh(**_MESH),
        compiler_params=_CP,
        scratch_types=[
            pltpu.VMEM((SEG,), jnp.int32),
            pltpu.VMEM((3, D, CW), jnp.float32),
            pltpu.VMEM((1, D, CH), jnp.float32),
            pltpu.VMEM((4, L, CH), jnp.float32),
            pltpu.SemaphoreType.DMA,
            pltpu.SemaphoreType.DMA,
        ],
    )(_sweep_body)
    mat = sweep(sent_pad, entT, tailT)
    mat2 = mat.reshape(MAXG * L, CH)

    nr = relation_embeddings.shape[0]
    rel2 = relation_embeddings.reshape(nr // 2, 2 * D)
    rrow = (relation >> 1).reshape(NW, nch, CH)
    rpar = (relation & 1).reshape(NW, nch, CH)

    score = functools.partial(
        pl.kernel,
        out_type=jax.ShapeDtypeStruct((b,), jnp.float32),
        mesh=plsc.VectorSubcoreMesh(**_MESH),
        compiler_params=_CP,
        scratch_types=[
            pltpu.VMEM((nch, CH), jnp.int32),
            pltpu.VMEM((nch, CH), jnp.int32),
            pltpu.VMEM((nch, CH), jnp.int32),
            pltpu.VMEM((nch, CH), jnp.int32),
            pltpu.VMEM((2, CH, CH), jnp.float32),
            pltpu.VMEM((2, CH, CH), jnp.float32),
            pltpu.VMEM((2, CH, 2 * D), jnp.float32),
            pltpu.VMEM((nch * CH,), jnp.float32),
            pltpu.SemaphoreType.DMA,
        ],
    )(functools.partial(_score_body, nch))
    return score(hpos, tpos, rrow, rpar, mat2, rel2)


# speedup vs baseline: 1.0071x; 1.0071x over previous
import functools

import jax
import jax.numpy as jnp
from jax import lax
from jax.experimental import pallas as pl
from jax.experimental.pallas import tpu as pltpu
from jax.experimental.pallas import tpu_sc as plsc

D = 64
NC = 2
NS = 16
NW = NC * NS
CH = 128
L = 16
CW = 512
NE = 1000000
NFULL = NE // CW
TAIL0 = NFULL * CW
BPW = 1024
GPW = BPW // L
MAXG = NW * GPW
SEG = BPW + L
UNROLL = 4

_CP = pltpu.CompilerParams(needs_layout_passes=False, use_tc_tiling_on_sc=True)
_MESH = dict(core_axis_name="c", subcore_axis_name="s")


def _rsqrt(x):
    i = plsc.bitcast(x, jnp.int32)
    i = jnp.int32(0x5F3759DF) - (i >> 1)
    y = plsc.bitcast(i, jnp.float32)
    hx = x * jnp.float32(0.5)
    for _ in range(3):
        y = y * (jnp.float32(1.5) - hx * y * y)
    return y


def _sweep_body(sent_hbm, entT_hbm, tailT_hbm, mat_hbm,
                sent_v, bbuf, tbuf, stage, semb, semf):
    wid = lax.axis_index("s") * NC + lax.axis_index("c")
    g0 = wid * GPW

    pltpu.sync_copy(
        sent_hbm.at[pl.ds(pl.multiple_of(wid * BPW, 8), SEG)], sent_v)

    iota = lax.iota(jnp.int32, L)

    e_first = sent_v[pl.ds(0, L)][0]
    e_last = sent_v[pl.ds(BPW - L, L)][L - 1]
    c_lo = e_first >> 9
    c_hi_all = e_last >> 9
    c_hi = jnp.minimum(c_hi_all, NFULL - 1)
    nck = jnp.maximum(c_hi - c_lo + 1, 0)

    def flush_group(g, fg):
        @pl.when(fg >= 2)
        def _():
            pltpu.make_async_copy(stage.at[0], mat_hbm.at[0], semf).wait()
        pltpu.async_copy(stage.at[g & 3], mat_hbm.at[g0 + g], semf)

    def _drain(fg3):
        @pl.when(fg3 >= 1)
        def _():
            pltpu.make_async_copy(stage.at[0], mat_hbm.at[0], semf).wait()

        @pl.when(fg3 >= 2)
        def _():
            pltpu.make_async_copy(stage.at[0], mat_hbm.at[0], semf).wait()

    def extract_chunk(cid, buf, slot_s, start, wmax, p, fg):
        slotv = jnp.full((L,), slot_s, jnp.int32)

        def cond(state):
            return state[2]

        def body(state):
            p_, fg_, _ = state
            ev = sent_v[pl.ds(p_, L)]
            cnt = plsc.all_reduce_population_count((ev >> 9) == cid)[0]
            cv = jnp.clip(ev - start, 0, wmax)
            tv = p_ + iota
            ssv = (tv >> 4) & 3
            rowv = tv & (L - 1)

            @pl.when(cnt > 0)
            def _():
                def jstep(i, _):
                    for u in range(UNROLL):
                        jv = (jnp.full((L,), i * UNROLL + u, jnp.int32)
                              + iota) & (D - 1)
                        g = plsc.load_gather(buf, [slotv, jv, cv])
                        plsc.store_scatter(stage, [ssv, rowv, jv], g)
                    return 0

                lax.fori_loop(0, D // UNROLL, jstep, 0)

            newp = p_ + cnt

            def doflush(fg2):
                flush_group(fg2, fg2)
                return fg2 + 1

            fg_ = lax.cond((newp >> 4) > fg_, doflush, lambda f: f, fg_)
            return (newp, fg_, (cnt == L) & (newp < BPW))

        p, fg, _ = lax.while_loop(cond, body, (p, fg, True))
        return p, fg

    def start_chunk(k, slot):
        off = pl.multiple_of((c_lo + k) * CW, CW)
        return pltpu.async_copy(
            entT_hbm.at[:, pl.ds(off, CW)], bbuf.at[slot], semb)

    @pl.when(nck > 0)
    def _():
        start_chunk(0, 0)

    @pl.when(nck > 1)
    def _():
        start_chunk(1, 1)

    def chunk_body(k, carry):
        p, fg = carry
        pltpu.make_async_copy(
            entT_hbm.at[:, pl.ds(0, CW)], bbuf.at[0], semb).wait()

        @pl.when(k + 2 < nck)
        def _():
            start_chunk(k + 2, (k + 2) % 3)

        return extract_chunk(c_lo + k, bbuf, k % 3, (c_lo + k) * CW,
                             CW - 1, p, fg)

    p, fg = lax.fori_loop(0, nck, chunk_body,
                          (jnp.int32(0), jnp.int32(0)))

    @pl.when(c_hi_all >= NFULL)
    def _():
        pltpu.sync_copy(tailT_hbm, tbuf.at[0])
        p2, fg2 = extract_chunk(jnp.int32(NFULL), tbuf, 0, TAIL0,
                                CH - 1, p, fg)
        del p2
        _drain(fg2)

    @pl.when(c_hi_all < NFULL)
    def _():
        _drain(fg)


def _score_body(nch, hpos_hbm, tpos_hbm, rrow_hbm, rpar_hbm, mat_hbm,
                rel_hbm, out_hbm, hidx, tidx, ridx, rpar, hbuf, tbuf, rbuf,
                sbuf, sem):
    wid = lax.axis_index("s") * NC + lax.axis_index("c")
    base = wid * (nch * CH)

    pltpu.sync_copy(hpos_hbm.at[wid], hidx)
    pltpu.sync_copy(tpos_hbm.at[wid], tidx)
    pltpu.sync_copy(rrow_hbm.at[wid], ridx)
    pltpu.sync_copy(rpar_hbm.at[wid], rpar)

    iota = lax.iota(jnp.int32, L)

    def gather(c, slot):
        return (
            pltpu.async_copy(mat_hbm.at[hidx.at[c]], hbuf.at[slot], sem),
            pltpu.async_copy(mat_hbm.at[tidx.at[c]], tbuf.at[slot], sem),
            pltpu.async_copy(rel_hbm.at[ridx.at[c]], rbuf.at[slot], sem),
        )

    cps = gather(0, 0)
    for c in range(nch):
        slot = c % 2
        for cp in cps:
            cp.wait()
        if c + 1 < nch:
            cps = gather(c + 1, 1 - slot)

        for g in range(CH // L):
            row = iota + g * L
            pr = rpar[c, pl.ds(g * L, L)] << 6
            zeros = jnp.zeros((L,), jnp.float32)

            def step(i, acc):
                sh, sr, st, xhr, xht, xrt = acc
                for u in range(UNROLL):
                    jv = (jnp.full((L,), i * UNROLL + u, jnp.int32)
                          + iota) & (D - 1)
                    gh = plsc.load_gather(hbuf.at[slot], [row, jv])
                    gt = plsc.load_gather(tbuf.at[slot], [row, jv])
                    gr = plsc.load_gather(rbuf.at[slot], [row, pr + jv])
                    sh = sh + gh * gh
                    sr = sr + gr * gr
                    st = st + gt * gt
                    xhr = xhr + gh * gr
                    xht = xht + gh * gt
                    xrt = xrt + gr * gt
                return (sh, sr, st, xhr, xht, xrt)

            sh, sr, st, xhr, xht, xrt = lax.fori_loop(
                0, D // UNROLL, step, (zeros,) * 6)

            cap = jnp.full((L,), 1e12, jnp.float32)
            ih = jnp.minimum(_rsqrt(sh), cap)
            ir = jnp.minimum(_rsqrt(sr), cap)
            it = jnp.minimum(_rsqrt(st), cap)
            s2 = (sh * ih * ih + sr * ir * ir + st * it * it
                  + jnp.float32(2.0) * (xhr * ih * ir - xht * ih * it
                                        - xrt * ir * it))
            s2 = jnp.maximum(s2, jnp.float32(0.0))
            sbuf[pl.ds(c * CH + g * L, L)] = s2 * _rsqrt(s2)

    pltpu.sync_copy(sbuf, out_hbm.at[pl.ds(base, nch * CH)])


def kernel(head, relation, tail, entity_embeddings, relation_embeddings):
    b = head.shape[0]
    nch = b // (NW * CH)

    ent_all = jnp.concatenate([head, tail])
    order = jnp.argsort(ent_all)
    inv_order = jnp.argsort(order).astype(jnp.int32)
    sorted_ent = ent_all[order]
    hpos = inv_order[:b].reshape(NW, nch, CH)
    tpos = inv_order[b:].reshape(NW, nch, CH)

    sent_pad = jnp.concatenate(
        [sorted_ent.astype(jnp.int32), jnp.zeros((SEG + 8,), jnp.int32)])

    entT = entity_embeddings.T
    tailT = jnp.pad(entity_embeddings[TAIL0:], ((0, CH - (NE - TAIL0)),
                                                (0, 0))).T

    sweep = functools.partial(
        pl.kernel,
        out_type=jax.ShapeDtypeStruct((MAXG, L, CH), jnp.float32),
        mesh=plsc.VectorSubcoreMesh(**_MESH),
        compiler_params=_CP,
        scratch_types=[
            pltpu.VMEM((SEG,), jnp.int32),
            pltpu.VMEM((3, D, CW), jnp.float32),
            pltpu.VMEM((1, D, CH), jnp.float32),
            pltpu.VMEM((4, L, CH), jnp.float32),
            pltpu.SemaphoreType.DMA,
            pltpu.SemaphoreType.DMA,
        ],
    )(_sweep_body)
    mat = sweep(sent_pad, entT, tailT)
    mat2 = mat.reshape(MAXG * L, CH)

    nr = relation_embeddings.shape[0]
    rel2 = relation_embeddings.reshape(nr // 2, 2 * D)
    rrow = (relation >> 1).reshape(NW, nch, CH)
    rpar = (relation & 1).reshape(NW, nch, CH)

    score = functools.partial(
        pl.kernel,
        out_type=jax.ShapeDtypeStruct((b,), jnp.float32),
        mesh=plsc.VectorSubcoreMesh(**_MESH),
        compiler_params=_CP,
        scratch_types=[
            pltpu.VMEM((nch, CH), jnp.int32),
            pltpu.VMEM((nch, CH), jnp.int32),
            pltpu.VMEM((nch, CH), jnp.int32),
            pltpu.VMEM((nch, CH), jnp.int32),
            pltpu.VMEM((2, CH, CH), jnp.float32),
            pltpu.VMEM((2, CH, CH), jnp.float32),
            pltpu.VMEM((2, CH, 2 * D), jnp.float32),
            pltpu.VMEM((nch * CH,), jnp.float32),
            pltpu.SemaphoreType.DMA,
        ],
    )(functools.partial(_score_body, nch))
    return score(hpos, tpos, rrow, rpar, mat2, rel2)


# speedup vs baseline: 1.0665x; 1.0590x over previous
import functools

import jax
import jax.numpy as jnp
from jax import lax
from jax.experimental import pallas as pl
from jax.experimental.pallas import tpu as pltpu
from jax.experimental.pallas import tpu_sc as plsc

D = 64
NC = 2
NS = 16
NW = NC * NS
CH = 128
L = 16
CW = 512
NE = 1000000
NFULL = NE // CW
TAIL0 = NFULL * CW
BPW = 1024
GPW = BPW // L
MAXG = NW * GPW
SEG = BPW + L
UNROLL = 4

_CP = pltpu.CompilerParams(needs_layout_passes=False, use_tc_tiling_on_sc=True)
_MESH = dict(core_axis_name="c", subcore_axis_name="s")


def _rsqrt(x):
    i = plsc.bitcast(x, jnp.int32)
    i = jnp.int32(0x5F3759DF) - (i >> 1)
    y = plsc.bitcast(i, jnp.float32)
    hx = x * jnp.float32(0.5)
    for _ in range(3):
        y = y * (jnp.float32(1.5) - hx * y * y)
    return y


def _sweep_body(sent_hbm, entT_hbm, tailT_hbm, mat_hbm,
                sent_v, bbuf, tbuf, stage, semb, semf):
    wid = lax.axis_index("s") * NC + lax.axis_index("c")
    g0 = wid * GPW

    pltpu.sync_copy(
        sent_hbm.at[pl.ds(pl.multiple_of(wid * BPW, 8), SEG)], sent_v)

    iota = lax.iota(jnp.int32, L)

    e_first = sent_v[pl.ds(0, L)][0]
    e_last = sent_v[pl.ds(BPW - L, L)][L - 1]
    c_lo = e_first >> 9
    c_hi_all = e_last >> 9
    c_hi = jnp.minimum(c_hi_all, NFULL - 1)
    nck = jnp.maximum(c_hi - c_lo + 1, 0)

    def flush_group(g, fg):
        @pl.when(fg >= 2)
        def _():
            pltpu.make_async_copy(stage.at[0], mat_hbm.at[0], semf).wait()
        pltpu.async_copy(stage.at[g & 3], mat_hbm.at[g0 + g], semf)

    def _drain(fg3):
        @pl.when(fg3 >= 1)
        def _():
            pltpu.make_async_copy(stage.at[0], mat_hbm.at[0], semf).wait()

        @pl.when(fg3 >= 2)
        def _():
            pltpu.make_async_copy(stage.at[0], mat_hbm.at[0], semf).wait()

    def extract_chunk(cid, buf, slot_s, start, wmax, p, fg):
        slotv = jnp.full((L,), slot_s, jnp.int32)

        def cond(state):
            return state[2]

        def body(state):
            p_, fg_, _ = state
            ev = sent_v[pl.ds(p_, L)]
            cnt = plsc.all_reduce_population_count((ev >> 9) == cid)[0]
            cv = jnp.clip(ev - start, 0, wmax)
            tv = p_ + iota
            ssv = (tv >> 4) & 3
            rowv = tv & (L - 1)

            @pl.when(cnt > 0)
            def _():
                def jstep(i, _):
                    for u in range(UNROLL):
                        jv = (jnp.full((L,), i * UNROLL + u, jnp.int32)
                              + iota) & (D - 1)
                        g = plsc.load_gather(buf, [slotv, jv, cv])
                        plsc.store_scatter(stage, [ssv, rowv, jv], g)
                    return 0

                lax.fori_loop(0, D // UNROLL, jstep, 0)

            newp = p_ + cnt

            def doflush(fg2):
                flush_group(fg2, fg2)
                return fg2 + 1

            fg_ = lax.cond((newp >> 4) > fg_, doflush, lambda f: f, fg_)
            return (newp, fg_, (cnt == L) & (newp < BPW))

        p, fg, _ = lax.while_loop(cond, body, (p, fg, True))
        return p, fg

    def start_chunk(k, slot):
        off = pl.multiple_of((c_lo + k) * CW, CW)
        return pltpu.async_copy(
            entT_hbm.at[:, pl.ds(off, CW)], bbuf.at[slot], semb)

    @pl.when(nck > 0)
    def _():
        start_chunk(0, 0)

    @pl.when(nck > 1)
    def _():
        start_chunk(1, 1)

    def chunk_body(k, carry):
        p, fg = carry
        pltpu.make_async_copy(
            entT_hbm.at[:, pl.ds(0, CW)], bbuf.at[0], semb).wait()

        @pl.when(k + 2 < nck)
        def _():
            start_chunk(k + 2, (k + 2) % 3)

        return extract_chunk(c_lo + k, bbuf, k % 3, (c_lo + k) * CW,
                             CW - 1, p, fg)

    p, fg = lax.fori_loop(0, nck, chunk_body,
                          (jnp.int32(0), jnp.int32(0)))

    @pl.when(c_hi_all >= NFULL)
    def _():
        pltpu.sync_copy(tailT_hbm, tbuf.at[0])
        p2, fg2 = extract_chunk(jnp.int32(NFULL), tbuf, 0, TAIL0,
                                CH - 1, p, fg)
        del p2
        _drain(fg2)

    @pl.when(c_hi_all < NFULL)
    def _():
        _drain(fg)


def _score_body(nch, hpos_hbm, tpos_hbm, rrow_hbm, rpar_hbm, mat_hbm,
                rel_hbm, out_hbm, hidx, tidx, ridx, rpar, hbuf, tbuf, rbuf,
                sbuf, sem):
    wid = lax.axis_index("s") * NC + lax.axis_index("c")
    base = wid * (nch * CH)

    pltpu.sync_copy(hpos_hbm.at[wid], hidx)
    pltpu.sync_copy(tpos_hbm.at[wid], tidx)
    pltpu.sync_copy(rrow_hbm.at[wid], ridx)
    pltpu.sync_copy(rpar_hbm.at[wid], rpar)

    iota = lax.iota(jnp.int32, L)

    def gather(c, slot):
        return (
            pltpu.async_copy(mat_hbm.at[hidx.at[c]], hbuf.at[slot], sem),
            pltpu.async_copy(mat_hbm.at[tidx.at[c]], tbuf.at[slot], sem),
            pltpu.async_copy(rel_hbm.at[ridx.at[c]], rbuf.at[slot], sem),
        )

    cps = gather(0, 0)
    for c in range(nch):
        slot = c % 2
        for cp in cps:
            cp.wait()
        if c + 1 < nch:
            cps = gather(c + 1, 1 - slot)

        for g in range(CH // L):
            row = iota + g * L
            pr = rpar[c, pl.ds(g * L, L)] << 6
            zeros = jnp.zeros((L,), jnp.float32)

            def step(i, acc):
                sh, sr, st, xhr, xht, xrt = acc
                for u in range(UNROLL):
                    jv = (jnp.full((L,), i * UNROLL + u, jnp.int32)
                          + iota) & (D - 1)
                    gh = plsc.load_gather(hbuf.at[slot], [row, jv])
                    gt = plsc.load_gather(tbuf.at[slot], [row, jv])
                    gr = plsc.load_gather(rbuf.at[slot], [row, pr + jv])
                    sh = sh + gh * gh
                    sr = sr + gr * gr
                    st = st + gt * gt
                    xhr = xhr + gh * gr
                    xht = xht + gh * gt
                    xrt = xrt + gr * gt
                return (sh, sr, st, xhr, xht, xrt)

            sh, sr, st, xhr, xht, xrt = lax.fori_loop(
                0, D // UNROLL, step, (zeros,) * 6)

            cap = jnp.full((L,), 1e12, jnp.float32)
            ih = jnp.minimum(_rsqrt(sh), cap)
            ir = jnp.minimum(_rsqrt(sr), cap)
            it = jnp.minimum(_rsqrt(st), cap)
            s2 = (sh * ih * ih + sr * ir * ir + st * it * it
                  + jnp.float32(2.0) * (xhr * ih * ir - xht * ih * it
                                        - xrt * ir * it))
            s2 = jnp.maximum(s2, jnp.float32(0.0))
            sbuf[pl.ds(c * CH + g * L, L)] = s2 * _rsqrt(s2)

    pltpu.sync_copy(sbuf, out_hbm.at[pl.ds(base, nch * CH)])


def kernel(head, relation, tail, entity_embeddings, relation_embeddings):
    b = head.shape[0]
    nch = b // (NW * CH)

    ent_all = jnp.concatenate([head, tail])
    sorted_ent, order = lax.sort_key_val(
        ent_all, jnp.arange(2 * b, dtype=jnp.int32))
    inv_order = jnp.argsort(order).astype(jnp.int32)
    hpos = inv_order[:b].reshape(NW, nch, CH)
    tpos = inv_order[b:].reshape(NW, nch, CH)

    sent_pad = jnp.concatenate(
        [sorted_ent.astype(jnp.int32), jnp.zeros((SEG + 8,), jnp.int32)])

    entT = entity_embeddings.T
    tailT = jnp.pad(entity_embeddings[TAIL0:], ((0, CH - (NE - TAIL0)),
                                                (0, 0))).T

    sweep = functools.partial(
        pl.kernel,
        out_type=jax.ShapeDtypeStruct((MAXG, L, CH), jnp.float32),
        mesh=plsc.VectorSubcoreMesh(**_MESH),
        compiler_params=_CP,
        scratch_types=[
            pltpu.VMEM((SEG,), jnp.int32),
            pltpu.VMEM((3, D, CW), jnp.float32),
            pltpu.VMEM((1, D, CH), jnp.float32),
            pltpu.VMEM((4, L, CH), jnp.float32),
            pltpu.SemaphoreType.DMA,
            pltpu.SemaphoreType.DMA,
        ],
    )(_sweep_body)
    mat = sweep(sent_pad, entT, tailT)
    mat2 = mat.reshape(MAXG * L, CH)

    nr = relation_embeddings.shape[0]
    rel2 = relation_embeddings.reshape(nr // 2, 2 * D)
    rrow = (relation >> 1).reshape(NW, nch, CH)
    rpar = (relation & 1).reshape(NW, nch, CH)

    score = functools.partial(
        pl.kernel,
        out_type=jax.ShapeDtypeStruct((b,), jnp.float32),
        mesh=plsc.VectorSubcoreMesh(**_MESH),
        compiler_params=_CP,
        scratch_types=[
            pltpu.VMEM((nch, CH), jnp.int32),
            pltpu.VMEM((nch, CH), jnp.int32),
            pltpu.VMEM((nch, CH), jnp.int32),
            pltpu.VMEM((nch, CH), jnp.int32),
            pltpu.VMEM((2, CH, CH), jnp.float32),
            pltpu.VMEM((2, CH, CH), jnp.float32),
            pltpu.VMEM((2, CH, 2 * D), jnp.float32),
            pltpu.VMEM((nch * CH,), jnp.float32),
            pltpu.SemaphoreType.DMA,
        ],
    )(functools.partial(_score_body, nch))
    return score(hpos, tpos, rrow, rpar, mat2, rel2)
